# baseline (speedup 1.0000x reference)
import functools

import jax
import jax.numpy as jnp
from jax import lax
from jax.experimental import pallas as pl
from jax.experimental.pallas import tpu as pltpu
from jax.experimental.pallas import tpu_sc as plsc

_NUM_CORES = 2
_NUM_SUBCORES = 16
_NW = _NUM_CORES * _NUM_SUBCORES

_CHUNK = 128
_D0 = 256
_DT = 128
_DW = _D0 + _DT


def _build_gather(n_idx: int, vocab: int):
    assert n_idx % (_NW * _CHUNK) == 0
    n_chunks = n_idx // _CHUNK
    chunks_per_w = n_chunks // _NW
    assert chunks_per_w % 2 == 0
    idx_per_w = n_idx // _NW

    mesh = plsc.VectorSubcoreMesh(
        core_axis_name="c", subcore_axis_name="s",
        num_cores=_NUM_CORES, num_subcores=_NUM_SUBCORES)

    @functools.partial(
        pl.kernel,
        out_type=jax.ShapeDtypeStruct((n_chunks, _CHUNK, _DW), jnp.float32),
        mesh=mesh,
        scratch_types=[
            pltpu.VMEM((idx_per_w,), jnp.int32),
            pltpu.VMEM((_CHUNK, _DW), jnp.float32),
            pltpu.VMEM((_CHUNK, _DW), jnp.float32),
            pltpu.SemaphoreType.DMA,
            pltpu.SemaphoreType.DMA,
            pltpu.SemaphoreType.DMA,
            pltpu.SemaphoreType.DMA,
            pltpu.SemaphoreType.DMA,
            pltpu.SemaphoreType.DMA,
        ],
    )
    def gather(table_hbm, tail_hbm, idx_hbm, out_hbm, idx_v, r0, r1,
               sa0, sb0, sw0, sa1, sb1, sw1):
        wid = lax.axis_index("s") * _NUM_CORES + lax.axis_index("c")
        cbase = wid * chunks_per_w
        pltpu.sync_copy(idx_hbm.at[wid], idx_v)

        bufs = (r0, r1)
        sas = (sa0, sa1)
        sbs = (sb0, sb1)
        sws = (sw0, sw1)

        def start_gather(c, j):
            ids = idx_v.at[pl.ds(c * _CHUNK, _CHUNK)]
            pltpu.async_copy(table_hbm.at[ids, pl.ds(0, _D0)],
                             bufs[j].at[:, pl.ds(0, _D0)], sas[j])
            pltpu.async_copy(tail_hbm.at[ids],
                             bufs[j].at[:, pl.ds(_D0, _DT)], sbs[j])

        def wait_gather(j):
            dummy = table_hbm.at[pl.ds(0, _CHUNK), pl.ds(0, _D0)]
            pltpu.make_async_copy(
                dummy, bufs[j].at[:, pl.ds(0, _D0)], sas[j]).wait()
            dummy_t = tail_hbm.at[pl.ds(0, _CHUNK)]
            pltpu.make_async_copy(
                dummy_t, bufs[j].at[:, pl.ds(_D0, _DT)], sbs[j]).wait()

        def start_write(c, j):
            pltpu.async_copy(bufs[j], out_hbm.at[cbase + c], sws[j])

        def wait_write(c, j):
            pltpu.make_async_copy(bufs[j], out_hbm.at[cbase + c],
                                  sws[j]).wait()

        start_gather(0, 0)
        start_gather(1, 1)

        @pl.loop(0, chunks_per_w, step=2)
        def _(c0):
            c1 = c0 + 1
            wait_gather(0)
            start_write(c0, 0)
            wait_gather(1)
            start_write(c1, 1)
            wait_write(c0, 0)

            @pl.when(c0 + 2 < chunks_per_w)
            def _():
                start_gather(c0 + 2, 0)

            wait_write(c1, 1)

            @pl.when(c1 + 2 < chunks_per_w)
            def _():
                start_gather(c1 + 2, 1)

    return gather


def kernel(x, table):
    b, l = x.shape
    vocab, dim = table.shape
    n_idx = b * l
    idx = x.reshape(_NW, n_idx // _NW).astype(jnp.int32)
    tail = table[:, dim - _DT:]
    wide = _build_gather(n_idx, vocab)(table, tail, idx)
    wide2 = wide.reshape(n_idx, _DW)
    seam = _D0 + (_D0 - (dim - _DT))
    embeddings = jnp.concatenate(
        [wide2[:, :_D0], wide2[:, seam:]], axis=1).reshape(b, l, dim)
    mask = jnp.ones((b, l), dtype=x.dtype)
    return (embeddings, mask)


# speedup vs baseline: 1.4056x; 1.4056x over previous
import functools

import jax
import jax.numpy as jnp
from jax import lax
from jax.experimental import pallas as pl
from jax.experimental.pallas import tpu as pltpu
from jax.experimental.pallas import tpu_sc as plsc

_NUM_CORES = 2
_NUM_SUBCORES = 16
_NW = _NUM_CORES * _NUM_SUBCORES

_CHUNK = 128
_D0 = 256
_DT = 128
_DW = _D0 + _DT


def _build_gather(n_idx: int, vocab: int):
    assert n_idx % (_NW * _CHUNK) == 0
    n_chunks = n_idx // _CHUNK
    chunks_per_w = n_chunks // _NW
    assert chunks_per_w % 2 == 0
    idx_per_w = n_idx // _NW

    mesh = plsc.VectorSubcoreMesh(
        core_axis_name="c", subcore_axis_name="s",
        num_cores=_NUM_CORES, num_subcores=_NUM_SUBCORES)

    @functools.partial(
        pl.kernel,
        out_type=jax.ShapeDtypeStruct((n_chunks, _CHUNK, _DW), jnp.float32),
        mesh=mesh,
        scratch_types=[
            pltpu.VMEM((idx_per_w,), jnp.int32),
            pltpu.VMEM((_CHUNK, _DW), jnp.float32),
            pltpu.VMEM((_CHUNK, _DW), jnp.float32),
            pltpu.SemaphoreType.DMA,
            pltpu.SemaphoreType.DMA,
            pltpu.SemaphoreType.DMA,
            pltpu.SemaphoreType.DMA,
            pltpu.SemaphoreType.DMA,
            pltpu.SemaphoreType.DMA,
        ],
    )
    def gather(table_hbm, tail_hbm, idx_hbm, out_hbm, idx_v, r0, r1,
               sa0, sb0, sw0, sa1, sb1, sw1):
        wid = lax.axis_index("s") * _NUM_CORES + lax.axis_index("c")
        cbase = wid * chunks_per_w
        pltpu.sync_copy(idx_hbm.at[wid], idx_v)

        bufs = (r0, r1)
        sas = (sa0, sa1)
        sbs = (sb0, sb1)
        sws = (sw0, sw1)

        def start_gather(c, j):
            ids = idx_v.at[pl.ds(c * _CHUNK, _CHUNK)]
            pltpu.async_copy(table_hbm.at[ids, pl.ds(0, _D0)],
                             bufs[j].at[:, pl.ds(0, _D0)], sas[j])
            pltpu.async_copy(tail_hbm.at[ids],
                             bufs[j].at[:, pl.ds(_D0, _DT)], sbs[j])

        def wait_gather(c, j):
            ids = idx_v.at[pl.ds(c * _CHUNK, _CHUNK)]
            pltpu.make_async_copy(table_hbm.at[ids, pl.ds(0, _D0)],
                                  bufs[j].at[:, pl.ds(0, _D0)], sas[j]).wait()
            pltpu.make_async_copy(tail_hbm.at[ids],
                                  bufs[j].at[:, pl.ds(_D0, _DT)],
                                  sbs[j]).wait()

        def start_write(c, j):
            pltpu.async_copy(bufs[j], out_hbm.at[cbase + c], sws[j])

        def wait_write(c, j):
            pltpu.make_async_copy(bufs[j], out_hbm.at[cbase + c],
                                  sws[j]).wait()

        start_gather(0, 0)
        start_gather(1, 1)

        @pl.loop(0, chunks_per_w, step=2)
        def _(c0):
            c1 = c0 + 1
            wait_gather(c0, 0)
            start_write(c0, 0)
            wait_gather(c1, 1)
            start_write(c1, 1)
            wait_write(c0, 0)

            @pl.when(c0 + 2 < chunks_per_w)
            def _():
                start_gather(c0 + 2, 0)

            wait_write(c1, 1)

            @pl.when(c1 + 2 < chunks_per_w)
            def _():
                start_gather(c1 + 2, 1)

    return gather


def kernel(x, table):
    b, l = x.shape
    vocab, dim = table.shape
    n_idx = b * l
    idx = x.reshape(_NW, n_idx // _NW).astype(jnp.int32)
    tail = jnp.pad(table[:, _D0:], ((0, 0), (0, _DT - (dim - _D0))))
    wide = _build_gather(n_idx, vocab)(table, tail, idx)
    embeddings = wide.reshape(n_idx, _DW)[:, :dim].reshape(b, l, dim)
    mask = jnp.ones((b, l), dtype=x.dtype)
    return (embeddings, mask)
